# baseline (speedup 1.0000x reference)
import functools

import jax
import jax.numpy as jnp
from jax import lax
from jax.experimental import pallas as pl
from jax.experimental.pallas import tpu as pltpu
from jax.experimental.pallas import tpu_sc as plsc

_NUM_EXPERTS = 16
_N_TOKENS = 16384
_SUPER = 1024
_NSUB = 4
_SUB = _SUPER // _NSUB
_NSLOT = 3

_NC = 2
_NS = 16
_NW = _NC * _NS
_ROWS_PER_W = _N_TOKENS // _NW


def _copy(h_hbm, buf, sem, block, slot, s):
    return pltpu.make_async_copy(
        h_hbm.at[pl.ds(block * _SUPER + s * _SUB, _SUB), :],
        buf.at[slot, pl.ds(s * _SUB, _SUB), :],
        sem.at[slot, s],
    )


def _issue(h_hbm, buf, sem, block, slot):
    for s in range(_NSUB):
        _copy(h_hbm, buf, sem, block, slot, s).start()


def _logits_kernel(h_hbm, w_ref, b_ref, out_ref, buf, sem):
    i = pl.program_id(0)
    nblocks = pl.num_programs(0)

    @pl.when(i == 0)
    def _warmup():
        for k in range(_NSLOT):
            _issue(h_hbm, buf, sem, k, k)

    slot = lax.rem(i, _NSLOT)
    for s in range(_NSUB):
        _copy(h_hbm, buf, sem, i, slot, s).wait()

    g = lax.dot_general(
        buf[slot], w_ref[...],
        dimension_numbers=(((1,), (1,)), ((), ())),
        preferred_element_type=jnp.float32,
    )
    out_ref[...] = g + b_ref[...]

    @pl.when(i + _NSLOT < nblocks)
    def _refill():
        _issue(h_hbm, buf, sem, i + _NSLOT, slot)


def _route_tile(g_v, o_v, t):
    base = t * 16
    rows = base + lax.iota(jnp.int32, 16)
    ge = [
        plsc.load_gather(g_v, [rows, jnp.full((16,), e, jnp.int32)])
        for e in range(_NUM_EXPERTS)
    ]
    neg_inf = jnp.full((16,), -jnp.inf, jnp.float32)
    zeros_i = jnp.zeros((16,), jnp.int32)
    m1 = ge[0]
    i1 = zeros_i
    for e in range(1, _NUM_EXPERTS):
        gt = ge[e] > m1
        m1 = jnp.where(gt, ge[e], m1)
        i1 = jnp.where(gt, e, i1)
    m2 = jnp.where(i1 == 0, neg_inf, ge[0])
    i2 = zeros_i
    for e in range(1, _NUM_EXPERTS):
        cand = jnp.where(i1 == e, neg_inf, ge[e])
        gt = cand > m2
        m2 = jnp.where(gt, cand, m2)
        i2 = jnp.where(gt, e, i2)
    r = jnp.exp(m2 - m1)
    w1 = 1.0 / (1.0 + r)
    w2 = r * w1
    zeros_f = jnp.zeros((16,), jnp.float32)
    for k in range(16):
        o_v[base + k] = zeros_f
    plsc.store_scatter(o_v, [rows, i1], w1)
    plsc.store_scatter(o_v, [rows, i2], w2)


def _route_kernel(g_hbm, out_hbm, g_v, o_v):
    wid = lax.axis_index("s") * _NC + lax.axis_index("c")
    base = wid * _ROWS_PER_W
    pltpu.sync_copy(g_hbm.at[pl.ds(base, _ROWS_PER_W), :], g_v)
    lax.fori_loop(
        0,
        _ROWS_PER_W // 16,
        lambda t, c: (_route_tile(g_v, o_v, t), c)[1],
        0,
    )
    pltpu.sync_copy(o_v, out_hbm.at[pl.ds(base, _ROWS_PER_W), :])


@functools.partial(jax.jit, static_argnames=())
def kernel(h, W, b):
    n, d = h.shape
    ne = W.shape[0]
    b2 = b.reshape(1, ne)
    g = pl.pallas_call(
        _logits_kernel,
        grid=(n // _SUPER,),
        in_specs=[
            pl.BlockSpec(memory_space=pl.ANY),
            pl.BlockSpec((ne, d), lambda i: (0, 0)),
            pl.BlockSpec((1, ne), lambda i: (0, 0)),
        ],
        out_specs=pl.BlockSpec((_SUPER, ne), lambda i: (i, 0)),
        out_shape=jax.ShapeDtypeStruct((n, ne), jnp.float32),
        scratch_shapes=[
            pltpu.VMEM((_NSLOT, _SUPER, 2048), jnp.float32),
            pltpu.SemaphoreType.DMA((_NSLOT, _NSUB)),
        ],
        compiler_params=pltpu.CompilerParams(
            dimension_semantics=("arbitrary",),
        ),
    )(h, W, b2)

    route = functools.partial(
        pl.kernel,
        out_type=jax.ShapeDtypeStruct((n, ne), jnp.float32),
        mesh=plsc.VectorSubcoreMesh(core_axis_name="c", subcore_axis_name="s"),
        compiler_params=pltpu.CompilerParams(needs_layout_passes=False),
        scratch_types=[
            pltpu.VMEM((_ROWS_PER_W, _NUM_EXPERTS), jnp.float32),
            pltpu.VMEM((_ROWS_PER_W, _NUM_EXPERTS), jnp.float32),
        ],
    )(_route_kernel)
    return route(g)


# speedup vs baseline: 1.0318x; 1.0318x over previous
import functools

import jax
import jax.numpy as jnp
from jax import lax
from jax.experimental import pallas as pl
from jax.experimental.pallas import tpu as pltpu
from jax.experimental.pallas import tpu_sc as plsc

_NUM_EXPERTS = 16
_N_TOKENS = 16384
_SUPER = 1024
_NSUB = 4
_SUB = _SUPER // _NSUB
_NSLOT = 3

_NC = 2
_NS = 16
_NW = _NC * _NS
_ROWS_PER_W = _N_TOKENS // _NW


def _copy(h_hbm, buf, sem, block, slot, s):
    return pltpu.make_async_copy(
        h_hbm.at[pl.ds(block * _SUPER + s * _SUB, _SUB), :],
        buf.at[slot, pl.ds(s * _SUB, _SUB), :],
        sem.at[slot, s],
    )


def _issue(h_hbm, buf, sem, block, slot):
    for s in range(_NSUB):
        _copy(h_hbm, buf, sem, block, slot, s).start()


def _logits_kernel(h_hbm, w_ref, b_ref, out_ref, buf, sem):
    i = pl.program_id(0)
    nblocks = pl.num_programs(0)

    @pl.when(i == 0)
    def _warmup():
        for k in range(_NSLOT):
            _issue(h_hbm, buf, sem, k, k)

    slot = lax.rem(i, _NSLOT)
    for s in range(_NSUB):
        _copy(h_hbm, buf, sem, i, slot, s).wait()

    g = lax.dot_general(
        buf[slot], w_ref[...],
        dimension_numbers=(((1,), (1,)), ((), ())),
        preferred_element_type=jnp.float32,
    )
    out_ref[...] = g + b_ref[...]

    @pl.when(i + _NSLOT < nblocks)
    def _refill():
        _issue(h_hbm, buf, sem, i + _NSLOT, slot)


def _route_one_row(g_v, o_v, idx, lane_t, lane_s, r):
    g = g_v[r]
    s = lax.sort(g)
    m1 = s.at[lane_t].get(mode="promise_in_bounds")
    m2 = s.at[lane_s].get(mode="promise_in_bounds")
    i1 = plsc.all_reduce_ffs(g == m1)
    at1 = idx == i1
    i2 = plsc.all_reduce_ffs((g == m2) & jnp.logical_not(at1))
    mask = at1 | (idx == i2)
    e = jnp.exp(g - m1)
    denom = 1.0 + jnp.exp(m2 - m1)
    o_v[r] = jnp.where(mask, e / denom, 0.0)


def _route_kernel(g_hbm, out_hbm, g_v, o_v):
    wid = lax.axis_index("s") * _NC + lax.axis_index("c")
    base = wid * _ROWS_PER_W
    pltpu.sync_copy(g_hbm.at[pl.ds(base, _ROWS_PER_W), :], g_v)
    idx = lax.iota(jnp.int32, _NUM_EXPERTS)
    lane_t = jnp.full((_NUM_EXPERTS,), _NUM_EXPERTS - 1, jnp.int32)
    lane_s = jnp.full((_NUM_EXPERTS,), _NUM_EXPERTS - 2, jnp.int32)

    @plsc.parallel_loop(0, _ROWS_PER_W, unroll=4)
    def _rows(r):
        _route_one_row(g_v, o_v, idx, lane_t, lane_s, r)

    pltpu.sync_copy(o_v, out_hbm.at[pl.ds(base, _ROWS_PER_W), :])


@functools.partial(jax.jit, static_argnames=())
def kernel(h, W, b):
    n, d = h.shape
    ne = W.shape[0]
    b2 = b.reshape(1, ne)
    g = pl.pallas_call(
        _logits_kernel,
        grid=(n // _SUPER,),
        in_specs=[
            pl.BlockSpec(memory_space=pl.ANY),
            pl.BlockSpec((ne, d), lambda i: (0, 0)),
            pl.BlockSpec((1, ne), lambda i: (0, 0)),
        ],
        out_specs=pl.BlockSpec((_SUPER, ne), lambda i: (i, 0)),
        out_shape=jax.ShapeDtypeStruct((n, ne), jnp.float32),
        scratch_shapes=[
            pltpu.VMEM((_NSLOT, _SUPER, 2048), jnp.float32),
            pltpu.SemaphoreType.DMA((_NSLOT, _NSUB)),
        ],
        compiler_params=pltpu.CompilerParams(
            dimension_semantics=("arbitrary",),
        ),
    )(h, W, b2)

    route = functools.partial(
        pl.kernel,
        out_type=jax.ShapeDtypeStruct((n, ne), jnp.float32),
        mesh=plsc.VectorSubcoreMesh(core_axis_name="c", subcore_axis_name="s"),
        compiler_params=pltpu.CompilerParams(needs_layout_passes=False),
        scratch_types=[
            pltpu.VMEM((_ROWS_PER_W, _NUM_EXPERTS), jnp.float32),
            pltpu.VMEM((_ROWS_PER_W, _NUM_EXPERTS), jnp.float32),
        ],
    )(_route_kernel)
    return route(g)
